# baseline (speedup 1.0000x reference)
import functools

import jax
import jax.numpy as jnp
from jax import lax
from jax.experimental import pallas as pl
from jax.experimental.pallas import tpu as pltpu
from jax.experimental.pallas import tpu_sc as plsc

_LANES = 16
_WIN = 128


def _lse_body(x_ref, o_ref):
    x = x_ref[...]
    o_ref[0] = jnp.log(jnp.sum(jnp.exp(x), axis=-1))


def _sc_body(rows_w, c, x_tab, lab_hbm, w_hbm, labv, idxv, winv, sem):
    nc = 2
    wid = lax.axis_index("s") * nc + lax.axis_index("c")
    base = wid * rows_w
    pltpu.sync_copy(lab_hbm.at[pl.ds(base, rows_w)], labv)

    iota16 = lax.broadcasted_iota(jnp.int32, (_LANES,), 0)
    cw = c // _WIN
    for j in range(rows_w // _LANES):
        lab = labv[pl.ds(j * _LANES, _LANES)]
        rows = (base + j * _LANES) + iota16
        idx = rows * cw + lax.shift_right_logical(lab, 7)
        idxv[j // 8, pl.ds((j % 8) * _LANES, _LANES)] = idx

    for j in range(rows_w // 128):
        pltpu.async_copy(x_tab.at[idxv.at[j]],
                         winv.at[pl.ds(j * 128, 128)], sem).wait()

    pltpu.sync_copy(winv, w_hbm.at[pl.ds(base, rows_w)])


def _combine_body(y_ref, l_ref, w_ref, o_ref):
    y = y_ref[...]
    lse = l_ref[...]
    w = w_ref[...]
    lane = jnp.maximum(y, 0) & (_WIN - 1)
    lane3 = jax.lax.broadcast_in_dim(lane, w.shape, (0, 1))
    iota3 = jax.lax.broadcasted_iota(jnp.int32, w.shape, 2)
    g = jnp.sum(jnp.where(iota3 == lane3, w, 0.0), axis=-1)
    loss = jnp.where(y != -1, lse - g, 0.0)
    o_ref[...] = jnp.sum(loss, axis=-1, keepdims=True)


def kernel(y_true, y_pred):
    b, a, c = y_pred.shape
    bbb = 64
    nblk = b // bbb
    nw = 32
    rows = b * a
    rows_w = rows // nw

    yi = y_true.astype(jnp.int32)
    lab_flat = jnp.maximum(yi.reshape(rows), 0)

    x_tab = y_pred.reshape(rows * (c // _WIN), _WIN)
    mesh = plsc.VectorSubcoreMesh(core_axis_name="c", subcore_axis_name="s")
    wins = pl.kernel(
        functools.partial(_sc_body, rows_w, c),
        out_type=jax.ShapeDtypeStruct((rows, _WIN), jnp.float32),
        mesh=mesh,
        scratch_types=[
            pltpu.VMEM((rows_w,), jnp.int32),
            pltpu.VMEM((rows_w // 128, 128), jnp.int32),
            pltpu.VMEM((rows_w, _WIN), jnp.float32),
            pltpu.SemaphoreType.DMA,
        ],
    )(x_tab, lab_flat)

    lse = pl.pallas_call(
        _lse_body,
        grid=(nblk,),
        in_specs=[pl.BlockSpec((bbb, a, c), lambda i: (i, 0, 0))],
        out_specs=pl.BlockSpec((1, bbb, a), lambda i: (i, 0, 0)),
        out_shape=jax.ShapeDtypeStruct((nblk, bbb, a), jnp.float32),
    )(y_pred)

    out = pl.pallas_call(
        _combine_body,
        in_specs=[
            pl.BlockSpec((b, a), lambda: (0, 0)),
            pl.BlockSpec((b, a), lambda: (0, 0)),
            pl.BlockSpec((b, a, _WIN), lambda: (0, 0, 0)),
        ],
        out_specs=pl.BlockSpec((b, 1), lambda: (0, 0)),
        out_shape=jax.ShapeDtypeStruct((b, 1), jnp.float32),
    )(yi, lse.reshape(b, a), wins.reshape(b, a, _WIN))

    return out.reshape(b)


# speedup vs baseline: 3.7142x; 3.7142x over previous
import jax
import jax.numpy as jnp
from jax import lax
from jax.experimental import pallas as pl


def _body(y_ref, x_ref, o_ref):
    x = x_ref[...]
    y = y_ref[...]
    lse = jnp.log(jnp.sum(jnp.exp(x), axis=-1))
    lab = jnp.maximum(y, 0)
    iota3 = lax.broadcasted_iota(jnp.int32, x.shape, 2)
    lab3 = lax.broadcast_in_dim(lab, x.shape, (0, 1))
    g = jnp.sum(jnp.where(iota3 == lab3, x, 0.0), axis=-1)
    loss = jnp.where(y != -1, lse - g, 0.0)
    o_ref[...] = jnp.sum(loss, axis=-1, keepdims=True)


def kernel(y_true, y_pred):
    b, a, c = y_pred.shape
    bbb = 64
    nblk = b // bbb
    out = pl.pallas_call(
        _body,
        grid=(nblk,),
        in_specs=[
            pl.BlockSpec((bbb, a), lambda i: (i, 0)),
            pl.BlockSpec((bbb, a, c), lambda i: (i, 0, 0)),
        ],
        out_specs=pl.BlockSpec((bbb, 1), lambda i: (i, 0)),
        out_shape=jax.ShapeDtypeStruct((b, 1), jnp.float32),
    )(y_true.astype(jnp.int32), y_pred)
    return out.reshape(b)


# speedup vs baseline: 4.1987x; 1.1304x over previous
import jax
import jax.numpy as jnp
from jax import lax
from jax.experimental import pallas as pl


def _body(y_ref, x_ref, o_ref):
    x = x_ref[...]
    y = y_ref[...]
    lse = jnp.log(jnp.sum(jnp.exp(x), axis=-1))
    lab = jnp.maximum(y, 0)
    iota3 = lax.broadcasted_iota(jnp.int32, x.shape, 2)
    lab3 = lax.broadcast_in_dim(lab, x.shape, (0, 1))
    g = jnp.sum(jnp.where(iota3 == lab3, x, 0.0), axis=-1)
    loss = jnp.where(y != -1, lse - g, 0.0)
    o_ref[...] = jnp.sum(loss, axis=-1, keepdims=True)


def kernel(y_true, y_pred):
    b, a, c = y_pred.shape
    bbb = 128
    nblk = b // bbb
    out = pl.pallas_call(
        _body,
        grid=(nblk,),
        in_specs=[
            pl.BlockSpec((bbb, a), lambda i: (i, 0)),
            pl.BlockSpec((bbb, a, c), lambda i: (i, 0, 0)),
        ],
        out_specs=pl.BlockSpec((bbb, 1), lambda i: (i, 0)),
        out_shape=jax.ShapeDtypeStruct((b, 1), jnp.float32),
    )(y_true.astype(jnp.int32), y_pred)
    return out.reshape(b)


# speedup vs baseline: 4.2519x; 1.0127x over previous
import jax
import jax.numpy as jnp
from jax import lax
from jax.experimental import pallas as pl


def _body(y_ref, x_ref, o_ref):
    x = x_ref[...]
    y = y_ref[...]
    lse = jnp.log(jnp.sum(jnp.exp(x), axis=-1))
    lab = jnp.maximum(y, 0)
    iota3 = lax.broadcasted_iota(jnp.int32, x.shape, 2)
    lab3 = lax.broadcast_in_dim(lab, x.shape, (0, 1))
    g = jnp.sum(jnp.where(iota3 == lab3, x, 0.0), axis=-1)
    loss = jnp.where(y != -1, lse - g, 0.0)
    o_ref[...] = jnp.sum(loss, axis=-1, keepdims=True)


def kernel(y_true, y_pred):
    b, a, c = y_pred.shape
    bbb = 256
    nblk = b // bbb
    out = pl.pallas_call(
        _body,
        grid=(nblk,),
        in_specs=[
            pl.BlockSpec((bbb, a), lambda i: (i, 0)),
            pl.BlockSpec((bbb, a, c), lambda i: (i, 0, 0)),
        ],
        out_specs=pl.BlockSpec((bbb, 1), lambda i: (i, 0)),
        out_shape=jax.ShapeDtypeStruct((b, 1), jnp.float32),
    )(y_true.astype(jnp.int32), y_pred)
    return out.reshape(b)
